# baseline (speedup 1.0000x reference)
import functools

import jax
import jax.numpy as jnp
from jax import lax
from jax.experimental import pallas as pl
from jax.experimental.pallas import tpu as pltpu
from jax.experimental.pallas import tpu_sc as plsc

T = 8192
D = 2048
E = 8
K = 2
FF = 8192

BT = 1024
NB = (T * K) // BT + E
NPAD = NB * BT
BF = 512
NF = FF // BF

_NC, _NS = 2, 16
NW = _NC * _NS


def _router_body(x_ref, wg1_ref, bg1_ref, wg2_ref, bg2_ref, wg3_ref,
                 logits_ref, idx_ref, w_ref):
    h = jnp.maximum(jnp.dot(x_ref[...], wg1_ref[...],
                            preferred_element_type=jnp.float32)
                    + bg1_ref[...][None, :], 0.0)
    h = jnp.maximum(jnp.dot(h, wg2_ref[...],
                            preferred_element_type=jnp.float32)
                    + bg2_ref[...][None, :], 0.0)
    lg = jnp.dot(h, wg3_ref[...], preferred_element_type=jnp.float32)
    logits_ref[...] = lg
    i0 = jnp.argmax(lg, axis=1).astype(jnp.int32)
    l0 = jnp.max(lg, axis=1)
    cols = lax.broadcasted_iota(jnp.int32, lg.shape, 1)
    masked = jnp.where(cols == i0[:, None], -jnp.inf, lg)
    i1 = jnp.argmax(masked, axis=1).astype(jnp.int32)
    l1 = jnp.max(masked, axis=1)
    w0 = jax.nn.sigmoid(l0 - l1)
    idx_ref[...] = jnp.stack([i0, i1], axis=1)
    w_ref[...] = jnp.stack([w0, 1.0 - w0], axis=1)


BTR = 1024


def _router(x, Wg1, bg1, Wg2, bg2, Wg3, interpret=False):
    btr = BTR
    grid = (T // btr,)
    return pl.pallas_call(
        _router_body,
        grid=grid,
        in_specs=[
            pl.BlockSpec((btr, D), lambda i: (i, 0)),
            pl.BlockSpec((D, D // 4), lambda i: (0, 0)),
            pl.BlockSpec((D // 4,), lambda i: (0,)),
            pl.BlockSpec((D // 4, D // 16), lambda i: (0, 0)),
            pl.BlockSpec((D // 16,), lambda i: (0,)),
            pl.BlockSpec((D // 16, E), lambda i: (0, 0)),
        ],
        out_specs=[
            pl.BlockSpec((btr, E), lambda i: (i, 0)),
            pl.BlockSpec((btr, K), lambda i: (i, 0)),
            pl.BlockSpec((btr, K), lambda i: (i, 0)),
        ],
        out_shape=[
            jax.ShapeDtypeStruct((T, E), jnp.float32),
            jax.ShapeDtypeStruct((T, K), jnp.int32),
            jax.ShapeDtypeStruct((T, K), jnp.float32),
        ],
        interpret=interpret,
    )(x, Wg1, bg1, Wg2, bg2, Wg3)


def _dispatch_meta(top_idx, top_w):
    e_flat = top_idx.reshape(-1)
    onehot = (e_flat[:, None] == jnp.arange(E)[None, :]).astype(jnp.int32)
    q = jnp.take_along_axis(jnp.cumsum(onehot, axis=0) - 1,
                            e_flat[:, None], axis=1)[:, 0]
    counts = jnp.sum(onehot, axis=0)
    nb = (counts + BT - 1) // BT
    bb = jnp.cumsum(nb)
    pstart = (bb - nb) * BT
    p_flat = pstart[e_flat] + q
    tok = jnp.arange(T * K, dtype=jnp.int32) // K
    tok_padded = jnp.zeros((NPAD,), jnp.int32).at[p_flat].set(tok)
    w_padded = jnp.zeros((NPAD,), jnp.float32).at[p_flat].set(top_w.reshape(-1))
    block_expert = jnp.clip(
        jnp.searchsorted(bb, jnp.arange(NB), side="right"),
        0, E - 1).astype(jnp.int32)
    pos = p_flat.astype(jnp.int32).reshape(T, K)
    return tok_padded, w_padded, block_expert, pos


_GCH = 48
_G_PER_W = NPAD // NW
_GN = _G_PER_W // _GCH
_D2 = D // 2


def _sc_gather(x, tok_padded):
    mesh = plsc.VectorSubcoreMesh(core_axis_name="c", subcore_axis_name="s")

    @functools.partial(
        pl.kernel, mesh=mesh,
        out_type=jax.ShapeDtypeStruct((NPAD, _D2), jnp.int32),
        scratch_types=[
            pltpu.VMEM((2, _GCH), jnp.int32),
            pltpu.VMEM((_GCH, _D2), jnp.int32),
            pltpu.VMEM((_GCH, _D2), jnp.int32),
            pltpu.SemaphoreType.DMA,
            pltpu.SemaphoreType.DMA,
            pltpu.SemaphoreType.DMA,
            pltpu.SemaphoreType.DMA,
        ],
    )
    def k(x_hbm, idx_hbm, out_hbm, idx_v, r0, r1, g0, g1, s0, s1):
        wid = lax.axis_index("s") * _NC + lax.axis_index("c")
        base = wid * _G_PER_W
        rows = (r0, r1)
        gsem = (g0, g1)
        ssem = (s0, s1)

        for c in range(_GN):
            b = c & 1
            off = base + c * _GCH
            if c >= 2:
                poff = base + (c - 2) * _GCH
                pltpu.make_async_copy(
                    rows[b], out_hbm.at[pl.ds(poff, _GCH)], ssem[b]).wait()
            pltpu.sync_copy(idx_hbm.at[pl.ds(off, _GCH)], idx_v.at[b])
            pltpu.async_copy(x_hbm.at[idx_v.at[b]], rows[b], gsem[b])
            if c >= 1:
                bp = 1 - b
                poff = base + (c - 1) * _GCH
                pltpu.make_async_copy(
                    x_hbm.at[idx_v.at[bp]], rows[bp], gsem[bp]).wait()
                pltpu.async_copy(
                    rows[bp], out_hbm.at[pl.ds(poff, _GCH)], ssem[bp])
        bl = (_GN - 1) & 1
        loff = base + (_GN - 1) * _GCH
        pltpu.make_async_copy(x_hbm.at[idx_v.at[bl]], rows[bl], gsem[bl]).wait()
        pltpu.async_copy(rows[bl], out_hbm.at[pl.ds(loff, _GCH)], ssem[bl])
        pltpu.make_async_copy(
            rows[1 - bl], out_hbm.at[pl.ds(loff - _GCH, _GCH)],
            ssem[1 - bl]).wait()
        pltpu.make_async_copy(
            rows[bl], out_hbm.at[pl.ds(loff, _GCH)], ssem[bl]).wait()

    return k(x, tok_padded)


def _ffn_body(be_ref, x_ref, w1_ref, b1_ref, w2_ref, b2_ref, wrow_ref,
              out_ref):
    f = pl.program_id(1)
    xb = x_ref[...]
    h = jnp.maximum(jnp.dot(xb, w1_ref[0],
                            preferred_element_type=jnp.float32)
                    + b1_ref[0], 0.0)
    part = jnp.dot(h.astype(jnp.bfloat16), w2_ref[0],
                   preferred_element_type=jnp.float32)

    @pl.when(f == 0)
    def _init():
        out_ref[...] = part

    @pl.when(f > 0)
    def _acc():
        out_ref[...] += part

    @pl.when(f == NF - 1)
    def _fini():
        w = wrow_ref[0, 0, :]
        out_ref[...] = (out_ref[...] + b2_ref[0]) * w[:, None]


def _ffn(x_sorted, W1, B1, W2, B2, w_padded, block_expert, interpret=False):
    grid_spec = pltpu.PrefetchScalarGridSpec(
        num_scalar_prefetch=1,
        grid=(NB, NF),
        in_specs=[
            pl.BlockSpec((BT, D), lambda i, f, be: (i, 0)),
            pl.BlockSpec((1, D, BF), lambda i, f, be: (be[i], 0, f)),
            pl.BlockSpec((1, 1, BF), lambda i, f, be: (be[i], 0, f)),
            pl.BlockSpec((1, BF, D), lambda i, f, be: (be[i], f, 0)),
            pl.BlockSpec((1, 1, D), lambda i, f, be: (be[i], 0, 0)),
            pl.BlockSpec((1, 1, BT), lambda i, f, be: (i, 0, 0)),
        ],
        out_specs=pl.BlockSpec((BT, D), lambda i, f, be: (i, 0)),
    )
    return pl.pallas_call(
        _ffn_body,
        grid_spec=grid_spec,
        out_shape=jax.ShapeDtypeStruct((NPAD, D), jnp.float32),
        compiler_params=pltpu.CompilerParams(
            dimension_semantics=("arbitrary", "arbitrary")),
        interpret=interpret,
    )(block_expert, x_sorted, W1.astype(jnp.bfloat16), B1.reshape(E, 1, FF),
      W2.astype(jnp.bfloat16), B2.reshape(E, 1, D),
      w_padded.reshape(NB, 1, BT))


_CCH = 16
_C_PER_W = T // NW
_LC = 16


def _sc_combine(y, pos0, pos1):
    mesh = plsc.VectorSubcoreMesh(core_axis_name="c", subcore_axis_name="s")

    @functools.partial(
        pl.kernel, mesh=mesh,
        out_type=jax.ShapeDtypeStruct((T, D), jnp.float32),
        scratch_types=[
            pltpu.VMEM((_CCH,), jnp.int32),
            pltpu.VMEM((_CCH,), jnp.int32),
            pltpu.VMEM((_CCH, D), jnp.float32),
            pltpu.VMEM((_CCH, D), jnp.float32),
            pltpu.SemaphoreType.DMA,
        ],
    )
    def k(y_hbm, p0_hbm, p1_hbm, out_hbm, i0_v, i1_v, y0_v, y1_v, sem):
        wid = lax.axis_index("s") * _NC + lax.axis_index("c")
        base = wid * _C_PER_W

        def chunk(c, _):
            off = base + c * _CCH
            pltpu.sync_copy(p0_hbm.at[pl.ds(off, _CCH)], i0_v)
            pltpu.sync_copy(p1_hbm.at[pl.ds(off, _CCH)], i1_v)
            pltpu.async_copy(y_hbm.at[i0_v], y0_v, sem).wait()
            pltpu.async_copy(y_hbm.at[i1_v], y1_v, sem).wait()

            def row(r, _):
                def col(kk, _):
                    for u in range(8):
                        sl = pl.ds((kk * 8 + u) * _LC, _LC)
                        y0_v[r, sl] += y1_v[r, sl]
                    return _
                lax.fori_loop(0, D // _LC // 8, col, None)
                return _

            lax.fori_loop(0, _CCH, row, None)
            pltpu.sync_copy(y0_v, out_hbm.at[pl.ds(off, _CCH)])
            return _

        lax.fori_loop(0, _C_PER_W // _CCH, chunk, None)

    return k(y, pos0, pos1)


def kernel(hidden_states, Wg1, bg1, Wg2, bg2, Wg3, W1, B1, W2, B2):
    x = hidden_states
    router_logits, top_idx, top_w = _router(x, Wg1, bg1, Wg2, bg2, Wg3)
    tok_padded, w_padded, block_expert, pos = _dispatch_meta(top_idx, top_w)
    xi32 = lax.bitcast_convert_type(
        x.astype(jnp.bfloat16).reshape(T, _D2, 2), jnp.int32)
    x_sorted = lax.bitcast_convert_type(
        _sc_gather(xi32, tok_padded), jnp.bfloat16).reshape(NPAD, D)
    y = _ffn(x_sorted, W1, B1, W2, B2, w_padded, block_expert)
    final = _sc_combine(y, pos[:, 0], pos[:, 1])
    return (final, router_logits)


# speedup vs baseline: 1.4263x; 1.4263x over previous
import functools

import jax
import jax.numpy as jnp
from jax import lax
from jax.experimental import pallas as pl
from jax.experimental.pallas import tpu as pltpu
from jax.experimental.pallas import tpu_sc as plsc

T = 8192
D = 2048
E = 8
K = 2
FF = 8192

BT = 512
NB = (T * K) // BT + E
NPAD = NB * BT
BF = 512
NF = FF // BF

_NC, _NS = 2, 16
NW = _NC * _NS


def _dispatch_meta(top_idx, top_w):
    e_flat = top_idx.reshape(-1)
    onehot = (e_flat[:, None] == jnp.arange(E)[None, :]).astype(jnp.int32)
    q = jnp.take_along_axis(jnp.cumsum(onehot, axis=0) - 1,
                            e_flat[:, None], axis=1)[:, 0]
    counts = jnp.sum(onehot, axis=0)
    nb = (counts + BT - 1) // BT
    bb = jnp.cumsum(nb)
    pstart = (bb - nb) * BT
    p_flat = pstart[e_flat] + q
    tok = jnp.arange(T * K, dtype=jnp.int32) // K
    tok_padded = jnp.zeros((NPAD,), jnp.int32).at[p_flat].set(tok)
    w_padded = jnp.zeros((NPAD,), jnp.float32).at[p_flat].set(top_w.reshape(-1))
    block_expert = jnp.clip(
        jnp.searchsorted(bb, jnp.arange(NB), side="right"),
        0, E - 1).astype(jnp.int32)
    pos = p_flat.astype(jnp.int32).reshape(T, K)
    return tok_padded, w_padded, block_expert, pos


_GCH = 16
_G_PER_W = NPAD // NW
_GN = _G_PER_W // _GCH


def _sc_gather(x, tok_padded):
    mesh = plsc.VectorSubcoreMesh(core_axis_name="c", subcore_axis_name="s")

    @functools.partial(
        pl.kernel, mesh=mesh,
        out_type=jax.ShapeDtypeStruct((NPAD, D), jnp.float32),
        scratch_types=[
            pltpu.VMEM((2, _GCH), jnp.int32),
            pltpu.VMEM((_GCH, D), jnp.float32),
            pltpu.VMEM((_GCH, D), jnp.float32),
            pltpu.SemaphoreType.DMA,
            pltpu.SemaphoreType.DMA,
            pltpu.SemaphoreType.DMA,
            pltpu.SemaphoreType.DMA,
        ],
    )
    def k(x_hbm, idx_hbm, out_hbm, idx_v, r0, r1, g0, g1, s0, s1):
        wid = lax.axis_index("s") * _NC + lax.axis_index("c")
        base = wid * _G_PER_W
        rows = (r0, r1)
        gsem = (g0, g1)
        ssem = (s0, s1)

        for c in range(_GN):
            b = c & 1
            off = base + c * _GCH
            if c >= 2:
                poff = base + (c - 2) * _GCH
                pltpu.make_async_copy(
                    rows[b], out_hbm.at[pl.ds(poff, _GCH)], ssem[b]).wait()
            pltpu.sync_copy(idx_hbm.at[pl.ds(off, _GCH)], idx_v.at[b])
            pltpu.async_copy(x_hbm.at[idx_v.at[b]], rows[b], gsem[b])
            if c >= 1:
                bp = 1 - b
                poff = base + (c - 1) * _GCH
                pltpu.make_async_copy(
                    x_hbm.at[idx_v.at[bp]], rows[bp], gsem[bp]).wait()
                pltpu.async_copy(
                    rows[bp], out_hbm.at[pl.ds(poff, _GCH)], ssem[bp])
        bl = (_GN - 1) & 1
        loff = base + (_GN - 1) * _GCH
        pltpu.make_async_copy(x_hbm.at[idx_v.at[bl]], rows[bl], gsem[bl]).wait()
        pltpu.async_copy(rows[bl], out_hbm.at[pl.ds(loff, _GCH)], ssem[bl])
        pltpu.make_async_copy(
            rows[1 - bl], out_hbm.at[pl.ds(loff - _GCH, _GCH)],
            ssem[1 - bl]).wait()
        pltpu.make_async_copy(
            rows[bl], out_hbm.at[pl.ds(loff, _GCH)], ssem[bl]).wait()

    return k(x, tok_padded)


def _ffn_body(be_ref, x_ref, w1_ref, b1_ref, w2_ref, b2_ref, wrow_ref,
              out_ref):
    f = pl.program_id(1)
    xb = x_ref[...].astype(jnp.bfloat16)
    h = jnp.maximum(jnp.dot(xb, w1_ref[0],
                            preferred_element_type=jnp.float32)
                    + b1_ref[0], 0.0)
    part = jnp.dot(h.astype(jnp.bfloat16), w2_ref[0],
                   preferred_element_type=jnp.float32)

    @pl.when(f == 0)
    def _init():
        out_ref[...] = part

    @pl.when(f > 0)
    def _acc():
        out_ref[...] += part

    @pl.when(f == NF - 1)
    def _fini():
        w = wrow_ref[0, 0, :]
        out_ref[...] = (out_ref[...] + b2_ref[0]) * w[:, None]


def _ffn(x_sorted, W1, B1, W2, B2, w_padded, block_expert, interpret=False):
    grid_spec = pltpu.PrefetchScalarGridSpec(
        num_scalar_prefetch=1,
        grid=(NB, NF),
        in_specs=[
            pl.BlockSpec((BT, D), lambda i, f, be: (i, 0)),
            pl.BlockSpec((1, D, BF), lambda i, f, be: (be[i], 0, f)),
            pl.BlockSpec((1, 1, BF), lambda i, f, be: (be[i], 0, f)),
            pl.BlockSpec((1, BF, D), lambda i, f, be: (be[i], f, 0)),
            pl.BlockSpec((1, 1, D), lambda i, f, be: (be[i], 0, 0)),
            pl.BlockSpec((1, 1, BT), lambda i, f, be: (i, 0, 0)),
        ],
        out_specs=pl.BlockSpec((BT, D), lambda i, f, be: (i, 0)),
    )
    return pl.pallas_call(
        _ffn_body,
        grid_spec=grid_spec,
        out_shape=jax.ShapeDtypeStruct((NPAD, D), jnp.float32),
        compiler_params=pltpu.CompilerParams(
            dimension_semantics=("arbitrary", "arbitrary")),
        interpret=interpret,
    )(block_expert, x_sorted, W1.astype(jnp.bfloat16), B1.reshape(E, 1, FF),
      W2.astype(jnp.bfloat16), B2.reshape(E, 1, D),
      w_padded.reshape(NB, 1, BT))


_CCH = 8
_C_PER_W = T // NW
_CN = _C_PER_W // _CCH
_LC = 16


def _sc_combine(y, pos0, pos1):
    mesh = plsc.VectorSubcoreMesh(core_axis_name="c", subcore_axis_name="s")

    @functools.partial(
        pl.kernel, mesh=mesh,
        out_type=jax.ShapeDtypeStruct((T, D), jnp.float32),
        scratch_types=[
            pltpu.VMEM((2, _CCH), jnp.int32),
            pltpu.VMEM((2, _CCH), jnp.int32),
            pltpu.VMEM((_CCH, D), jnp.float32),
            pltpu.VMEM((_CCH, D), jnp.float32),
            pltpu.VMEM((_CCH, D), jnp.float32),
            pltpu.VMEM((_CCH, D), jnp.float32),
            pltpu.SemaphoreType.DMA,
            pltpu.SemaphoreType.DMA,
            pltpu.SemaphoreType.DMA,
            pltpu.SemaphoreType.DMA,
            pltpu.SemaphoreType.DMA,
            pltpu.SemaphoreType.DMA,
        ],
    )
    def k(y_hbm, p0_hbm, p1_hbm, out_hbm, i0_v, i1_v,
          a0, a1, b0, b1, g00, g01, g10, g11, ss0, ss1):
        wid = lax.axis_index("s") * _NC + lax.axis_index("c")
        base = wid * _C_PER_W
        ya = (a0, a1)
        yb = (b0, b1)
        g0 = (g00, g01)
        g1 = (g10, g11)
        ss = (ss0, ss1)

        def add_rows(dst, src):
            def row(r, _):
                def col(kk, _):
                    for u in range(8):
                        sl = pl.ds((kk * 8 + u) * _LC, _LC)
                        dst[r, sl] += src[r, sl]
                    return _
                lax.fori_loop(0, D // _LC // 8, col, None)
                return _
            lax.fori_loop(0, _CCH, row, None)

        for c in range(_CN):
            b = c & 1
            off = base + c * _CCH
            if c >= 2:
                poff = base + (c - 2) * _CCH
                pltpu.make_async_copy(
                    ya[b], out_hbm.at[pl.ds(poff, _CCH)], ss[b]).wait()
            pltpu.sync_copy(p0_hbm.at[pl.ds(off, _CCH)], i0_v.at[b])
            pltpu.sync_copy(p1_hbm.at[pl.ds(off, _CCH)], i1_v.at[b])
            pltpu.async_copy(y_hbm.at[i0_v.at[b]], ya[b], g0[b])
            pltpu.async_copy(y_hbm.at[i1_v.at[b]], yb[b], g1[b])
            if c >= 1:
                bp = 1 - b
                poff = base + (c - 1) * _CCH
                pltpu.make_async_copy(y_hbm.at[i0_v.at[bp]], ya[bp],
                                      g0[bp]).wait()
                pltpu.make_async_copy(y_hbm.at[i1_v.at[bp]], yb[bp],
                                      g1[bp]).wait()
                add_rows(ya[bp], yb[bp])
                pltpu.async_copy(
                    ya[bp], out_hbm.at[pl.ds(poff, _CCH)], ss[bp])
        bl = (_CN - 1) & 1
        loff = base + (_CN - 1) * _CCH
        pltpu.make_async_copy(y_hbm.at[i0_v.at[bl]], ya[bl], g0[bl]).wait()
        pltpu.make_async_copy(y_hbm.at[i1_v.at[bl]], yb[bl], g1[bl]).wait()
        add_rows(ya[bl], yb[bl])
        pltpu.async_copy(ya[bl], out_hbm.at[pl.ds(loff, _CCH)], ss[bl])
        pltpu.make_async_copy(
            ya[1 - bl], out_hbm.at[pl.ds(loff - _CCH, _CCH)],
            ss[1 - bl]).wait()
        pltpu.make_async_copy(
            ya[bl], out_hbm.at[pl.ds(loff, _CCH)], ss[bl]).wait()

    return k(y, pos0, pos1)


def kernel(hidden_states, Wg1, bg1, Wg2, bg2, Wg3, W1, B1, W2, B2):
    x = hidden_states
    gh = jnp.maximum(x @ Wg1 + bg1, 0.0)
    gh = jnp.maximum(gh @ Wg2 + bg2, 0.0)
    router_logits = gh @ Wg3
    routing_weights = jax.nn.softmax(router_logits.astype(jnp.float32), axis=1)
    top_w, top_idx = jax.lax.top_k(routing_weights, K)
    top_w = top_w / jnp.sum(top_w, axis=-1, keepdims=True)
    top_idx = top_idx.astype(jnp.int32)
    tok_padded, w_padded, block_expert, pos = _dispatch_meta(top_idx, top_w)
    x_sorted = _sc_gather(x, tok_padded)
    y = _ffn(x_sorted, W1, B1, W2, B2, w_padded, block_expert)
    final = _sc_combine(y, pos[:, 0], pos[:, 1])
    return (final, router_logits)


# speedup vs baseline: 1.5489x; 1.0860x over previous
import functools

import jax
import jax.numpy as jnp
from jax import lax
from jax.experimental import pallas as pl
from jax.experimental.pallas import tpu as pltpu
from jax.experimental.pallas import tpu_sc as plsc

T = 8192
D = 2048
E = 8
K = 2
FF = 8192

BT = 512
NB = (T * K) // BT + E
NPAD = NB * BT
BF = 1024
NF = FF // BF

_NC, _NS = 2, 16
NW = _NC * _NS


def _dispatch_meta(top_idx, top_w):
    e_flat = top_idx.reshape(-1)
    onehot = (e_flat[:, None] == jnp.arange(E)[None, :]).astype(jnp.int32)
    q = jnp.take_along_axis(jnp.cumsum(onehot, axis=0) - 1,
                            e_flat[:, None], axis=1)[:, 0]
    counts = jnp.sum(onehot, axis=0)
    nb = (counts + BT - 1) // BT
    bb = jnp.cumsum(nb)
    pstart = (bb - nb) * BT
    p_flat = pstart[e_flat] + q
    tok = jnp.arange(T * K, dtype=jnp.int32) // K
    tok_padded = jnp.zeros((NPAD,), jnp.int32).at[p_flat].set(tok)
    w_padded = jnp.zeros((NPAD,), jnp.float32).at[p_flat].set(top_w.reshape(-1))
    block_expert = jnp.clip(
        jnp.searchsorted(bb, jnp.arange(NB), side="right"),
        0, E - 1).astype(jnp.int32)
    pos = p_flat.astype(jnp.int32).reshape(T, K)
    return tok_padded, w_padded, block_expert, pos


_GCH = 16
_G_PER_W = NPAD // NW
_GN = _G_PER_W // _GCH


def _sc_gather(x, tok_padded):
    mesh = plsc.VectorSubcoreMesh(core_axis_name="c", subcore_axis_name="s")

    @functools.partial(
        pl.kernel, mesh=mesh,
        out_type=jax.ShapeDtypeStruct((NPAD, D), jnp.float32),
        scratch_types=[
            pltpu.VMEM((2, _GCH), jnp.int32),
            pltpu.VMEM((_GCH, D), jnp.float32),
            pltpu.VMEM((_GCH, D), jnp.float32),
            pltpu.SemaphoreType.DMA,
            pltpu.SemaphoreType.DMA,
            pltpu.SemaphoreType.DMA,
            pltpu.SemaphoreType.DMA,
        ],
    )
    def k(x_hbm, idx_hbm, out_hbm, idx_v, r0, r1, g0, g1, s0, s1):
        wid = lax.axis_index("s") * _NC + lax.axis_index("c")
        base = wid * _G_PER_W
        rows = (r0, r1)
        gsem = (g0, g1)
        ssem = (s0, s1)

        for c in range(_GN):
            b = c & 1
            off = base + c * _GCH
            if c >= 2:
                poff = base + (c - 2) * _GCH
                pltpu.make_async_copy(
                    rows[b], out_hbm.at[pl.ds(poff, _GCH)], ssem[b]).wait()
            pltpu.sync_copy(idx_hbm.at[pl.ds(off, _GCH)], idx_v.at[b])
            pltpu.async_copy(x_hbm.at[idx_v.at[b]], rows[b], gsem[b])
            if c >= 1:
                bp = 1 - b
                poff = base + (c - 1) * _GCH
                pltpu.make_async_copy(
                    x_hbm.at[idx_v.at[bp]], rows[bp], gsem[bp]).wait()
                pltpu.async_copy(
                    rows[bp], out_hbm.at[pl.ds(poff, _GCH)], ssem[bp])
        bl = (_GN - 1) & 1
        loff = base + (_GN - 1) * _GCH
        pltpu.make_async_copy(x_hbm.at[idx_v.at[bl]], rows[bl], gsem[bl]).wait()
        pltpu.async_copy(rows[bl], out_hbm.at[pl.ds(loff, _GCH)], ssem[bl])
        pltpu.make_async_copy(
            rows[1 - bl], out_hbm.at[pl.ds(loff - _GCH, _GCH)],
            ssem[1 - bl]).wait()
        pltpu.make_async_copy(
            rows[bl], out_hbm.at[pl.ds(loff, _GCH)], ssem[bl]).wait()

    return k(x, tok_padded)


def _ffn_body(be_ref, x_ref, w1_ref, b1_ref, w2_ref, b2_ref, wrow_ref,
              out_ref):
    f = pl.program_id(1)
    xb = x_ref[...].astype(jnp.bfloat16)
    h = jnp.maximum(jnp.dot(xb, w1_ref[0],
                            preferred_element_type=jnp.float32)
                    + b1_ref[0], 0.0)
    part = jnp.dot(h.astype(jnp.bfloat16), w2_ref[0],
                   preferred_element_type=jnp.float32)

    @pl.when(f == 0)
    def _init():
        out_ref[...] = part

    @pl.when(f > 0)
    def _acc():
        out_ref[...] += part

    @pl.when(f == NF - 1)
    def _fini():
        w = wrow_ref[0, 0, :]
        out_ref[...] = (out_ref[...] + b2_ref[0]) * w[:, None]


def _ffn(x_sorted, W1, B1, W2, B2, w_padded, block_expert, interpret=False):
    grid_spec = pltpu.PrefetchScalarGridSpec(
        num_scalar_prefetch=1,
        grid=(NB, NF),
        in_specs=[
            pl.BlockSpec((BT, D), lambda i, f, be: (i, 0)),
            pl.BlockSpec((1, D, BF), lambda i, f, be: (be[i], 0, f)),
            pl.BlockSpec((1, 1, BF), lambda i, f, be: (be[i], 0, f)),
            pl.BlockSpec((1, BF, D), lambda i, f, be: (be[i], f, 0)),
            pl.BlockSpec((1, 1, D), lambda i, f, be: (be[i], 0, 0)),
            pl.BlockSpec((1, 1, BT), lambda i, f, be: (i, 0, 0)),
        ],
        out_specs=pl.BlockSpec((BT, D), lambda i, f, be: (i, 0)),
    )
    return pl.pallas_call(
        _ffn_body,
        grid_spec=grid_spec,
        out_shape=jax.ShapeDtypeStruct((NPAD, D), jnp.float32),
        compiler_params=pltpu.CompilerParams(
            dimension_semantics=("arbitrary", "arbitrary")),
        interpret=interpret,
    )(block_expert, x_sorted, W1.astype(jnp.bfloat16), B1.reshape(E, 1, FF),
      W2.astype(jnp.bfloat16), B2.reshape(E, 1, D),
      w_padded.reshape(NB, 1, BT))


_CCH = 8
_C_PER_W = T // NW
_CN = _C_PER_W // _CCH
_LC = 16


def _sc_combine(y, pos0, pos1):
    mesh = plsc.VectorSubcoreMesh(core_axis_name="c", subcore_axis_name="s")

    @functools.partial(
        pl.kernel, mesh=mesh,
        out_type=jax.ShapeDtypeStruct((T, D), jnp.float32),
        scratch_types=[
            pltpu.VMEM((2, _CCH), jnp.int32),
            pltpu.VMEM((2, _CCH), jnp.int32),
            pltpu.VMEM((_CCH, D), jnp.float32),
            pltpu.VMEM((_CCH, D), jnp.float32),
            pltpu.VMEM((_CCH, D), jnp.float32),
            pltpu.VMEM((_CCH, D), jnp.float32),
            pltpu.SemaphoreType.DMA,
            pltpu.SemaphoreType.DMA,
            pltpu.SemaphoreType.DMA,
            pltpu.SemaphoreType.DMA,
            pltpu.SemaphoreType.DMA,
            pltpu.SemaphoreType.DMA,
        ],
    )
    def k(y_hbm, p0_hbm, p1_hbm, out_hbm, i0_v, i1_v,
          a0, a1, b0, b1, g00, g01, g10, g11, ss0, ss1):
        wid = lax.axis_index("s") * _NC + lax.axis_index("c")
        base = wid * _C_PER_W
        ya = (a0, a1)
        yb = (b0, b1)
        g0 = (g00, g01)
        g1 = (g10, g11)
        ss = (ss0, ss1)

        def add_rows(dst, src):
            def row(r, _):
                def col(kk, _):
                    for u in range(8):
                        sl = pl.ds((kk * 8 + u) * _LC, _LC)
                        dst[r, sl] += src[r, sl]
                    return _
                lax.fori_loop(0, D // _LC // 8, col, None)
                return _
            lax.fori_loop(0, _CCH, row, None)

        for c in range(_CN):
            b = c & 1
            off = base + c * _CCH
            if c >= 2:
                poff = base + (c - 2) * _CCH
                pltpu.make_async_copy(
                    ya[b], out_hbm.at[pl.ds(poff, _CCH)], ss[b]).wait()
            pltpu.sync_copy(p0_hbm.at[pl.ds(off, _CCH)], i0_v.at[b])
            pltpu.sync_copy(p1_hbm.at[pl.ds(off, _CCH)], i1_v.at[b])
            pltpu.async_copy(y_hbm.at[i0_v.at[b]], ya[b], g0[b])
            pltpu.async_copy(y_hbm.at[i1_v.at[b]], yb[b], g1[b])
            if c >= 1:
                bp = 1 - b
                poff = base + (c - 1) * _CCH
                pltpu.make_async_copy(y_hbm.at[i0_v.at[bp]], ya[bp],
                                      g0[bp]).wait()
                pltpu.make_async_copy(y_hbm.at[i1_v.at[bp]], yb[bp],
                                      g1[bp]).wait()
                add_rows(ya[bp], yb[bp])
                pltpu.async_copy(
                    ya[bp], out_hbm.at[pl.ds(poff, _CCH)], ss[bp])
        bl = (_CN - 1) & 1
        loff = base + (_CN - 1) * _CCH
        pltpu.make_async_copy(y_hbm.at[i0_v.at[bl]], ya[bl], g0[bl]).wait()
        pltpu.make_async_copy(y_hbm.at[i1_v.at[bl]], yb[bl], g1[bl]).wait()
        add_rows(ya[bl], yb[bl])
        pltpu.async_copy(ya[bl], out_hbm.at[pl.ds(loff, _CCH)], ss[bl])
        pltpu.make_async_copy(
            ya[1 - bl], out_hbm.at[pl.ds(loff - _CCH, _CCH)],
            ss[1 - bl]).wait()
        pltpu.make_async_copy(
            ya[bl], out_hbm.at[pl.ds(loff, _CCH)], ss[bl]).wait()

    return k(y, pos0, pos1)


def kernel(hidden_states, Wg1, bg1, Wg2, bg2, Wg3, W1, B1, W2, B2):
    x = hidden_states
    gh = jnp.maximum(x @ Wg1 + bg1, 0.0)
    gh = jnp.maximum(gh @ Wg2 + bg2, 0.0)
    router_logits = gh @ Wg3
    routing_weights = jax.nn.softmax(router_logits.astype(jnp.float32), axis=1)
    top_w, top_idx = jax.lax.top_k(routing_weights, K)
    top_w = top_w / jnp.sum(top_w, axis=-1, keepdims=True)
    top_idx = top_idx.astype(jnp.int32)
    tok_padded, w_padded, block_expert, pos = _dispatch_meta(top_idx, top_w)
    x_sorted = _sc_gather(x, tok_padded)
    y = _ffn(x_sorted, W1, B1, W2, B2, w_padded, block_expert)
    final = _sc_combine(y, pos[:, 0], pos[:, 1])
    return (final, router_logits)


# speedup vs baseline: 1.6075x; 1.0378x over previous
import functools

import jax
import jax.numpy as jnp
from jax import lax
from jax.experimental import pallas as pl
from jax.experimental.pallas import tpu as pltpu
from jax.experimental.pallas import tpu_sc as plsc

T = 8192
D = 2048
E = 8
K = 2
FF = 8192

BT = 512
NB = (T * K) // BT + E
NPAD = NB * BT
BF = 2048
NF = FF // BF

_NC, _NS = 2, 16
NW = _NC * _NS


def _dispatch_meta(top_idx, top_w):
    e_flat = top_idx.reshape(-1)
    onehot = (e_flat[:, None] == jnp.arange(E)[None, :]).astype(jnp.int32)
    q = jnp.take_along_axis(jnp.cumsum(onehot, axis=0) - 1,
                            e_flat[:, None], axis=1)[:, 0]
    counts = jnp.sum(onehot, axis=0)
    nb = (counts + BT - 1) // BT
    bb = jnp.cumsum(nb)
    pstart = (bb - nb) * BT
    p_flat = pstart[e_flat] + q
    tok = jnp.arange(T * K, dtype=jnp.int32) // K
    tok_padded = jnp.zeros((NPAD,), jnp.int32).at[p_flat].set(tok)
    w_padded = jnp.zeros((NPAD,), jnp.float32).at[p_flat].set(top_w.reshape(-1))
    block_expert = jnp.clip(
        jnp.searchsorted(bb, jnp.arange(NB), side="right"),
        0, E - 1).astype(jnp.int32)
    pos = p_flat.astype(jnp.int32).reshape(T, K)
    return tok_padded, w_padded, block_expert, pos


_GCH = 16
_G_PER_W = NPAD // NW
_GN = _G_PER_W // _GCH


def _sc_gather(x, tok_padded):
    mesh = plsc.VectorSubcoreMesh(core_axis_name="c", subcore_axis_name="s")

    @functools.partial(
        pl.kernel, mesh=mesh,
        out_type=jax.ShapeDtypeStruct((NPAD, D), jnp.float32),
        scratch_types=[
            pltpu.VMEM((2, _GCH), jnp.int32),
            pltpu.VMEM((_GCH, D), jnp.float32),
            pltpu.VMEM((_GCH, D), jnp.float32),
            pltpu.SemaphoreType.DMA,
            pltpu.SemaphoreType.DMA,
            pltpu.SemaphoreType.DMA,
            pltpu.SemaphoreType.DMA,
        ],
    )
    def k(x_hbm, idx_hbm, out_hbm, idx_v, r0, r1, g0, g1, s0, s1):
        wid = lax.axis_index("s") * _NC + lax.axis_index("c")
        base = wid * _G_PER_W
        rows = (r0, r1)
        gsem = (g0, g1)
        ssem = (s0, s1)

        for c in range(_GN):
            b = c & 1
            off = base + c * _GCH
            if c >= 2:
                poff = base + (c - 2) * _GCH
                pltpu.make_async_copy(
                    rows[b], out_hbm.at[pl.ds(poff, _GCH)], ssem[b]).wait()
            pltpu.sync_copy(idx_hbm.at[pl.ds(off, _GCH)], idx_v.at[b])
            pltpu.async_copy(x_hbm.at[idx_v.at[b]], rows[b], gsem[b])
            if c >= 1:
                bp = 1 - b
                poff = base + (c - 1) * _GCH
                pltpu.make_async_copy(
                    x_hbm.at[idx_v.at[bp]], rows[bp], gsem[bp]).wait()
                pltpu.async_copy(
                    rows[bp], out_hbm.at[pl.ds(poff, _GCH)], ssem[bp])
        bl = (_GN - 1) & 1
        loff = base + (_GN - 1) * _GCH
        pltpu.make_async_copy(x_hbm.at[idx_v.at[bl]], rows[bl], gsem[bl]).wait()
        pltpu.async_copy(rows[bl], out_hbm.at[pl.ds(loff, _GCH)], ssem[bl])
        pltpu.make_async_copy(
            rows[1 - bl], out_hbm.at[pl.ds(loff - _GCH, _GCH)],
            ssem[1 - bl]).wait()
        pltpu.make_async_copy(
            rows[bl], out_hbm.at[pl.ds(loff, _GCH)], ssem[bl]).wait()

    return k(x, tok_padded)


def _ffn_body(be_ref, x_ref, w1_ref, b1_ref, w2_ref, b2_ref, wrow_ref,
              out_ref):
    f = pl.program_id(1)
    xb = x_ref[...].astype(jnp.bfloat16)
    h = jnp.maximum(jnp.dot(xb, w1_ref[0],
                            preferred_element_type=jnp.float32)
                    + b1_ref[0], 0.0)
    part = jnp.dot(h.astype(jnp.bfloat16), w2_ref[0],
                   preferred_element_type=jnp.float32)

    @pl.when(f == 0)
    def _init():
        out_ref[...] = part

    @pl.when(f > 0)
    def _acc():
        out_ref[...] += part

    @pl.when(f == NF - 1)
    def _fini():
        w = wrow_ref[0, 0, :]
        out_ref[...] = (out_ref[...] + b2_ref[0]) * w[:, None]


def _ffn(x_sorted, W1, B1, W2, B2, w_padded, block_expert, interpret=False):
    grid_spec = pltpu.PrefetchScalarGridSpec(
        num_scalar_prefetch=1,
        grid=(NB, NF),
        in_specs=[
            pl.BlockSpec((BT, D), lambda i, f, be: (i, 0)),
            pl.BlockSpec((1, D, BF), lambda i, f, be: (be[i], 0, f)),
            pl.BlockSpec((1, 1, BF), lambda i, f, be: (be[i], 0, f)),
            pl.BlockSpec((1, BF, D), lambda i, f, be: (be[i], f, 0)),
            pl.BlockSpec((1, 1, D), lambda i, f, be: (be[i], 0, 0)),
            pl.BlockSpec((1, 1, BT), lambda i, f, be: (i, 0, 0)),
        ],
        out_specs=pl.BlockSpec((BT, D), lambda i, f, be: (i, 0)),
    )
    return pl.pallas_call(
        _ffn_body,
        grid_spec=grid_spec,
        out_shape=jax.ShapeDtypeStruct((NPAD, D), jnp.float32),
        compiler_params=pltpu.CompilerParams(
            dimension_semantics=("arbitrary", "arbitrary")),
        interpret=interpret,
    )(block_expert, x_sorted, W1.astype(jnp.bfloat16), B1.reshape(E, 1, FF),
      W2.astype(jnp.bfloat16), B2.reshape(E, 1, D),
      w_padded.reshape(NB, 1, BT))


_CCH = 8
_C_PER_W = T // NW
_CN = _C_PER_W // _CCH
_LC = 16


def _sc_combine(y, pos0, pos1):
    mesh = plsc.VectorSubcoreMesh(core_axis_name="c", subcore_axis_name="s")

    @functools.partial(
        pl.kernel, mesh=mesh,
        out_type=jax.ShapeDtypeStruct((T, D), jnp.float32),
        scratch_types=[
            pltpu.VMEM((2, _CCH), jnp.int32),
            pltpu.VMEM((2, _CCH), jnp.int32),
            pltpu.VMEM((_CCH, D), jnp.float32),
            pltpu.VMEM((_CCH, D), jnp.float32),
            pltpu.VMEM((_CCH, D), jnp.float32),
            pltpu.VMEM((_CCH, D), jnp.float32),
            pltpu.SemaphoreType.DMA,
            pltpu.SemaphoreType.DMA,
            pltpu.SemaphoreType.DMA,
            pltpu.SemaphoreType.DMA,
            pltpu.SemaphoreType.DMA,
            pltpu.SemaphoreType.DMA,
        ],
    )
    def k(y_hbm, p0_hbm, p1_hbm, out_hbm, i0_v, i1_v,
          a0, a1, b0, b1, g00, g01, g10, g11, ss0, ss1):
        wid = lax.axis_index("s") * _NC + lax.axis_index("c")
        base = wid * _C_PER_W
        ya = (a0, a1)
        yb = (b0, b1)
        g0 = (g00, g01)
        g1 = (g10, g11)
        ss = (ss0, ss1)

        def add_rows(dst, src):
            def row(r, _):
                def col(kk, _):
                    for u in range(8):
                        sl = pl.ds((kk * 8 + u) * _LC, _LC)
                        dst[r, sl] += src[r, sl]
                    return _
                lax.fori_loop(0, D // _LC // 8, col, None)
                return _
            lax.fori_loop(0, _CCH, row, None)

        for c in range(_CN):
            b = c & 1
            off = base + c * _CCH
            if c >= 2:
                poff = base + (c - 2) * _CCH
                pltpu.make_async_copy(
                    ya[b], out_hbm.at[pl.ds(poff, _CCH)], ss[b]).wait()
            pltpu.sync_copy(p0_hbm.at[pl.ds(off, _CCH)], i0_v.at[b])
            pltpu.sync_copy(p1_hbm.at[pl.ds(off, _CCH)], i1_v.at[b])
            pltpu.async_copy(y_hbm.at[i0_v.at[b]], ya[b], g0[b])
            pltpu.async_copy(y_hbm.at[i1_v.at[b]], yb[b], g1[b])
            if c >= 1:
                bp = 1 - b
                poff = base + (c - 1) * _CCH
                pltpu.make_async_copy(y_hbm.at[i0_v.at[bp]], ya[bp],
                                      g0[bp]).wait()
                pltpu.make_async_copy(y_hbm.at[i1_v.at[bp]], yb[bp],
                                      g1[bp]).wait()
                add_rows(ya[bp], yb[bp])
                pltpu.async_copy(
                    ya[bp], out_hbm.at[pl.ds(poff, _CCH)], ss[bp])
        bl = (_CN - 1) & 1
        loff = base + (_CN - 1) * _CCH
        pltpu.make_async_copy(y_hbm.at[i0_v.at[bl]], ya[bl], g0[bl]).wait()
        pltpu.make_async_copy(y_hbm.at[i1_v.at[bl]], yb[bl], g1[bl]).wait()
        add_rows(ya[bl], yb[bl])
        pltpu.async_copy(ya[bl], out_hbm.at[pl.ds(loff, _CCH)], ss[bl])
        pltpu.make_async_copy(
            ya[1 - bl], out_hbm.at[pl.ds(loff - _CCH, _CCH)],
            ss[1 - bl]).wait()
        pltpu.make_async_copy(
            ya[bl], out_hbm.at[pl.ds(loff, _CCH)], ss[bl]).wait()

    return k(y, pos0, pos1)


def kernel(hidden_states, Wg1, bg1, Wg2, bg2, Wg3, W1, B1, W2, B2):
    x = hidden_states
    gh = jnp.maximum(x @ Wg1 + bg1, 0.0)
    gh = jnp.maximum(gh @ Wg2 + bg2, 0.0)
    router_logits = gh @ Wg3
    routing_weights = jax.nn.softmax(router_logits.astype(jnp.float32), axis=1)
    top_w, top_idx = jax.lax.top_k(routing_weights, K)
    top_w = top_w / jnp.sum(top_w, axis=-1, keepdims=True)
    top_idx = top_idx.astype(jnp.int32)
    tok_padded, w_padded, block_expert, pos = _dispatch_meta(top_idx, top_w)
    x_sorted = _sc_gather(x, tok_padded)
    y = _ffn(x_sorted, W1, B1, W2, B2, w_padded, block_expert)
    final = _sc_combine(y, pos[:, 0], pos[:, 1])
    return (final, router_logits)
